# baseline (speedup 1.0000x reference)
import jax
import jax.numpy as jnp
from jax import lax
from jax.experimental import pallas as pl

NCLS = 1000
BR = 256


def _onehot_block(x_ref, o_ref):
    xb = x_ref[0, 0, :]
    iota = lax.broadcasted_iota(jnp.int32, (BR, NCLS), 1)
    o_ref[0] = (xb[:, None] == iota).astype(o_ref.dtype)


def kernel(x):
    out_dtype = jax.dtypes.canonicalize_dtype(jnp.int64)
    n, k = x.shape
    rows = n * k
    g = rows // BR
    xf = x.reshape(g, 1, BR).astype(jnp.int32)
    out = pl.pallas_call(
        _onehot_block,
        grid=(g,),
        in_specs=[pl.BlockSpec((1, 1, BR), lambda i: (i, 0, 0))],
        out_specs=pl.BlockSpec((1, BR, NCLS), lambda i: (i, 0, 0)),
        out_shape=jax.ShapeDtypeStruct((g, BR, NCLS), out_dtype),
    )(xf)
    return out.reshape(n, k, NCLS)


# speedup vs baseline: 1.7989x; 1.7989x over previous
import jax
import jax.numpy as jnp
from jax import lax
from jax.experimental import pallas as pl

NCLS = 1000
BN = 64


def _onehot_block(x_ref, o_ref):
    xb = x_ref[...]
    iota = lax.broadcasted_iota(jnp.int32, (xb.shape[0], xb.shape[1], NCLS), 2)
    o_ref[...] = (xb[:, :, None] == iota).astype(o_ref.dtype)


def kernel(x):
    out_dtype = jax.dtypes.canonicalize_dtype(jnp.int64)
    n, k = x.shape
    g = n // BN
    xi = x.astype(jnp.int32)
    return pl.pallas_call(
        _onehot_block,
        grid=(g,),
        in_specs=[pl.BlockSpec((BN, k), lambda i: (i, 0))],
        out_specs=pl.BlockSpec((BN, k, NCLS), lambda i: (i, 0, 0)),
        out_shape=jax.ShapeDtypeStruct((n, k, NCLS), out_dtype),
    )(xi)


# speedup vs baseline: 1.8072x; 1.0046x over previous
import jax
import jax.numpy as jnp
from jax import lax
from jax.experimental import pallas as pl

NCLS = 1000
BN = 128


def _onehot_block(x_ref, o_ref):
    xb = x_ref[...]
    iota = lax.broadcasted_iota(jnp.int32, (xb.shape[0], xb.shape[1], NCLS), 2)
    o_ref[...] = (xb[:, :, None] == iota).astype(o_ref.dtype)


def kernel(x):
    out_dtype = jax.dtypes.canonicalize_dtype(jnp.int64)
    n, k = x.shape
    g = n // BN
    xi = x.astype(jnp.int32)
    return pl.pallas_call(
        _onehot_block,
        grid=(g,),
        in_specs=[pl.BlockSpec((BN, k), lambda i: (i, 0))],
        out_specs=pl.BlockSpec((BN, k, NCLS), lambda i: (i, 0, 0)),
        out_shape=jax.ShapeDtypeStruct((n, k, NCLS), out_dtype),
    )(xi)


# speedup vs baseline: 7.9592x; 4.4041x over previous
import jax
import jax.numpy as jnp
from jax import lax
from jax.experimental import pallas as pl

NCLS = 1000
BJ = 2048


def _onehot_block(x_ref, o_ref):
    xrow = x_ref[0, 0, :]
    cls = lax.broadcasted_iota(jnp.int32, (NCLS, BJ), 0)
    o_ref[0] = (xrow[None, :] == cls).astype(o_ref.dtype)


def kernel(x):
    out_dtype = jax.dtypes.canonicalize_dtype(jnp.int64)
    n, k = x.shape
    nj = n // BJ
    xt = x.astype(jnp.int32).T.reshape(k, 1, n)
    out = pl.pallas_call(
        _onehot_block,
        grid=(k, nj),
        in_specs=[pl.BlockSpec((1, 1, BJ), lambda i, j: (i, 0, j))],
        out_specs=pl.BlockSpec((1, NCLS, BJ), lambda i, j: (i, 0, j)),
        out_shape=jax.ShapeDtypeStruct((k, NCLS, n), out_dtype),
    )(xt)
    return out.transpose(2, 0, 1)


# speedup vs baseline: 8.0698x; 1.0139x over previous
import jax
import jax.numpy as jnp
from jax import lax
from jax.experimental import pallas as pl

NCLS = 1000
BJ = 2048


def _onehot_block(x_ref, o_ref):
    i = pl.program_id(0)
    j = pl.program_id(1)
    xrow = x_ref[pl.ds(i, 1), pl.ds(j * BJ, BJ)]
    cls = lax.broadcasted_iota(jnp.int32, (NCLS, BJ), 0)
    o_ref[0] = (xrow == cls).astype(o_ref.dtype)


def kernel(x):
    out_dtype = jax.dtypes.canonicalize_dtype(jnp.int64)
    n, k = x.shape
    nj = n // BJ
    xt = x.astype(jnp.int32).T
    out = pl.pallas_call(
        _onehot_block,
        grid=(k, nj),
        in_specs=[pl.BlockSpec((k, n), lambda i, j: (0, 0))],
        out_specs=pl.BlockSpec((1, NCLS, BJ), lambda i, j: (i, 0, j)),
        out_shape=jax.ShapeDtypeStruct((k, NCLS, n), out_dtype),
    )(xt)
    return out.transpose(2, 0, 1)
